# baseline (speedup 1.0000x reference)
import functools

import numpy as np
import jax
import jax.numpy as jnp
from jax import lax
from jax.experimental import pallas as pl
from jax.experimental.pallas import tpu as pltpu
from jax.experimental.pallas import tpu_sc as plsc

_NB = 8
_SH = (128, 128, 128)
_N = 128 ** 3
_CHUNK = 4096
_NCHUNK = _N // _CHUNK
_ROWS = 256
_NC, _NS = 2, 16
_NW = _NC * _NS
_CPW = _NCHUNK // _NW


def _build_tables():
    D = 3
    total = D * _NB
    h = np.arange(_N, dtype=np.int64)
    gray = np.bitwise_xor(h, h >> 1)
    cols = []
    for dim in range(D):
        g = np.zeros_like(h)
        for bit in range(_NB):
            b = (gray >> (total - 1 - (bit * D + dim))) & 1
            g = g | (b << (_NB - 1 - bit))
        cols.append(g)
    for bit in range(_NB - 1, -1, -1):
        low = (1 << (_NB - 1 - bit)) - 1
        for dim in range(D - 1, -1, -1):
            mask = (cols[dim] >> (_NB - 1 - bit)) & 1
            cols[0] = np.bitwise_xor(cols[0], mask * low)
            to_flip = (1 - mask) * (np.bitwise_xor(cols[0], cols[dim]) & low)
            cols[dim] = np.bitwise_xor(cols[dim], to_flip)
            cols[0] = np.bitwise_xor(cols[0], to_flip)
    idx = np.zeros((_N,), dtype=np.int64)
    for d in range(D):
        idx = idx * _SH[d] + cols[d]
    idx = np.clip(idx, 0, _N - 1)

    rows = (idx // 16).reshape(_NCHUNK, _CHUNK)
    rowtab = np.empty((_NCHUNK, _ROWS), dtype=np.int32)
    loc = np.empty((_NCHUNK, _CHUNK), dtype=np.int32)
    for c in range(_NCHUNK):
        u, inv = np.unique(rows[c], return_inverse=True)
        assert len(u) == _ROWS
        rowtab[c] = u.astype(np.int32)
        loc[c] = (inv * 16 + (idx[c * _CHUNK:(c + 1) * _CHUNK] % 16)).astype(
            np.int32)
    return rowtab.reshape(_NCHUNK, 2, 128), loc


_ROWTAB_NP, _LOC_NP = _build_tables()

_mesh = plsc.VectorSubcoreMesh(core_axis_name="c", subcore_axis_name="s")


@functools.partial(
    pl.kernel,
    out_type=jax.ShapeDtypeStruct((_N,), jnp.float32),
    mesh=_mesh,
    compiler_params=pltpu.CompilerParams(needs_layout_passes=False,
                                         use_tc_tiling_on_sc=False),
    scratch_types=[
        pltpu.VMEM((_CPW, 2, 128), jnp.int32),
        pltpu.VMEM((4, _CHUNK), jnp.int32),
        pltpu.VMEM((4, _ROWS, 16), jnp.float32),
        pltpu.VMEM((4, _CHUNK), jnp.float32),
        pltpu.SemaphoreType.DMA((4,)),
        pltpu.SemaphoreType.DMA((4,)),
        pltpu.SemaphoreType.DMA((4,)),
    ],
)
def _hilbert_sc(x_hbm, rowtab_hbm, loc_hbm, out_hbm,
                rows_v, tab_v, blk_v, outb_v, sem_t, sem_g, sem_o):
    wid = lax.axis_index("s") * _NC + lax.axis_index("c")
    base = wid * _CPW
    pltpu.sync_copy(rowtab_hbm.at[wid], rows_v)

    def start_fetch(j):
        p = j % 4
        t = pltpu.async_copy(loc_hbm.at[base + j], tab_v.at[p], sem_t.at[p])
        g0 = pltpu.async_copy(x_hbm.at[rows_v.at[j, 0]],
                              blk_v.at[p, pl.ds(0, 128)], sem_g.at[p])
        g1 = pltpu.async_copy(x_hbm.at[rows_v.at[j, 1]],
                              blk_v.at[p, pl.ds(128, 128)], sem_g.at[p])
        return (t, g0, g1)

    fetches = [start_fetch(0), start_fetch(1), start_fetch(2), start_fetch(3)]
    stores = [None, None, None, None]
    for j in range(_CPW):
        p = j % 4
        for cp in fetches[p]:
            cp.wait()

        tab_p, blk_p, out_p = tab_v.at[p], blk_v.at[p], outb_v.at[p]

        @plsc.parallel_loop(0, _ROWS, step=4, unroll=4)
        def _permute(i):
            for u in range(4):
                lv = tab_p[pl.ds((i + u) * 16, 16)]
                r = lax.shift_right_logical(lv, 4)
                k = lax.bitwise_and(lv, 15)
                out_p[pl.ds((i + u) * 16, 16)] = plsc.load_gather(blk_p,
                                                                  [r, k])

        if stores[p] is not None:
            stores[p].wait()
        if j + 4 < _CPW:
            fetches[p] = start_fetch(j + 4)
        stores[p] = pltpu.async_copy(
            out_p, out_hbm.at[pl.ds((base + j) * _CHUNK, _CHUNK)], sem_o.at[p])
    for st in stores:
        st.wait()


def kernel(x):
    x2 = x.reshape(_N // 16, 16)
    rowtab = jnp.asarray(_ROWTAB_NP).reshape(_NW, _CPW, 2, 128)
    return _hilbert_sc(x2, rowtab, jnp.asarray(_LOC_NP))
